# baseline (speedup 1.0000x reference)
import dataclasses
import functools

import jax
import jax.numpy as jnp
from jax import lax
from jax.experimental import pallas as pl
from jax.experimental.pallas import tpu as pltpu
from jax.experimental.pallas import tpu_sc as plsc

N, D, K, A = 20000, 2048, 16, 8
ALPHA = 0.9
BETA = 1.0 - ALPHA
APPLY_PROB = 0.9
NSIZE = 1
L = 16

_COND = True
_R_KN = 13
_R_KA = 1
_R_KNI = 6

_scalar_mesh = plsc.ScalarSubcoreMesh(axis_name="c", num_cores=2)
_vector_mesh = plsc.VectorSubcoreMesh(core_axis_name="c", subcore_axis_name="s")

_cp = pltpu.CompilerParams()
if "needs_layout_passes" in pltpu.CompilerParams.__dataclass_fields__:
    _cp = dataclasses.replace(_cp, needs_layout_passes=False)

_cp_chase = dataclasses.replace(_cp, use_tc_tiling_on_sc=False)


@functools.partial(
    pl.kernel,
    out_type=jax.ShapeDtypeStruct((24,), jnp.int32),
    mesh=_scalar_mesh,
    compiler_params=_cp_chase,
    scratch_types=[
        pltpu.SMEM((K,), jnp.int32),
        pltpu.SMEM((A,), jnp.int32),
        pltpu.SMEM((1,), jnp.int32),
        pltpu.SMEM((24,), jnp.int32),
        pltpu.SemaphoreType.DMA,
    ],
)
def _chase_sc(cell_hbm, nns_hbm, mnn_hbm, o_hbm, nrow, mrow, cbuf, obuf, sem):
    @pl.when(lax.axis_index("c") == 0)
    def _():
        pltpu.async_copy(cell_hbm, cbuf, sem).wait()
        c = cbuf[0]
        pltpu.async_copy(mnn_hbm.at[c], mrow, sem).wait()
        anchor = mrow[_R_KA]
        cp_n1 = pltpu.async_copy(nns_hbm.at[c], nrow, sem)
        cp_n1.wait()
        n_intra = nrow[_R_KN]
        pltpu.async_copy(nns_hbm.at[anchor], nrow, sem).wait()
        n_inter = nrow[_R_KNI]
        obuf[0] = n_intra
        obuf[8] = anchor
        obuf[16] = n_inter
        pltpu.async_copy(obuf, o_hbm, sem).wait()


@functools.partial(
    pl.kernel,
    out_type=jax.ShapeDtypeStruct((2, D), jnp.float32),
    mesh=_vector_mesh,
    compiler_params=_cp,
    scratch_types=[
        pltpu.VMEM((24,), jnp.int32),
        pltpu.VMEM((1, D), jnp.float32),
        pltpu.VMEM((1, D), jnp.float32),
        pltpu.VMEM((D,), jnp.float32),
        pltpu.VMEM((1, D), jnp.float32),
        pltpu.SemaphoreType.DMA,
        pltpu.SemaphoreType.DMA,
    ],
)
def _interp_sc(x1_hbm, idx_hbm, X_hbm, o_hbm, iv, xa, xc, x1v, outv, s0, s1):
    core = lax.axis_index("c")
    sub = lax.axis_index("s")

    @pl.when((core == 0) & (sub == 0))
    def _():
        cp_x1 = pltpu.async_copy(x1_hbm, x1v, s0)
        pltpu.sync_copy(idx_hbm, iv)
        cp_xa = pltpu.async_copy(X_hbm.at[iv.at[pl.ds(0, 1)]], xa, s1)
        cp_x1.wait()

        @pl.loop(0, D, step=L)
        def _(i):
            outv[0, pl.ds(i, L)] = ALPHA * x1v[pl.ds(i, L)]

        cp_xa.wait()

        @pl.loop(0, D, step=L)
        def _(i):
            sl = pl.ds(i, L)
            outv[0, sl] = outv[0, sl] + BETA * xa[0, sl]

        pltpu.sync_copy(outv, o_hbm.at[pl.ds(0, 1)])

    @pl.when((core == 1) & (sub == 0))
    def _():
        pltpu.sync_copy(idx_hbm, iv)
        cp_xb = pltpu.async_copy(X_hbm.at[iv.at[pl.ds(8, 1)]], xa, s0)
        cp_xc = pltpu.async_copy(X_hbm.at[iv.at[pl.ds(16, 1)]], xc, s1)
        cp_xb.wait()

        @pl.loop(0, D, step=L)
        def _(i):
            outv[0, pl.ds(i, L)] = ALPHA * xa[0, pl.ds(i, L)]

        cp_xc.wait()

        @pl.loop(0, D, step=L)
        def _(i):
            sl = pl.ds(i, L)
            outv[0, sl] = outv[0, sl] + BETA * xc[0, sl]

        pltpu.sync_copy(outv, o_hbm.at[pl.ds(1, 1)])


def kernel(x1, x2, cell_ids, X, nns_idx, mnn_idx):
    if _COND:
        cell = cell_ids.astype(jnp.int32).reshape(1)
        idx = _chase_sc(cell, nns_idx, mnn_idx)
        return _interp_sc(x1, idx, X)
    else:
        return jnp.stack([x1, x2])


# speedup vs baseline: 2.0342x; 2.0342x over previous
import dataclasses
import functools

import jax
import jax.numpy as jnp
from jax import lax
from jax.experimental import pallas as pl
from jax.experimental.pallas import tpu as pltpu
from jax.experimental.pallas import tpu_sc as plsc

N, D, K, A = 20000, 2048, 16, 8
ALPHA = 0.9
BETA = 1.0 - ALPHA
APPLY_PROB = 0.9
NSIZE = 1
L = 16

_COND = True
_R_KN = 13
_R_KA = 1
_R_KNI = 6

_scalar_mesh = plsc.ScalarSubcoreMesh(axis_name="c", num_cores=2)
_vector_mesh = plsc.VectorSubcoreMesh(core_axis_name="c", subcore_axis_name="s")

_cp = pltpu.CompilerParams()
if "needs_layout_passes" in pltpu.CompilerParams.__dataclass_fields__:
    _cp = dataclasses.replace(_cp, needs_layout_passes=False)


@functools.partial(
    pl.kernel,
    out_type=jax.ShapeDtypeStruct((24,), jnp.int32),
    mesh=_scalar_mesh,
    scratch_types=[
        pltpu.SMEM((4, 128), jnp.int32),
        pltpu.SMEM((4, 128), jnp.int32),
        pltpu.SMEM((1,), jnp.int32),
        pltpu.SMEM((24,), jnp.int32),
        pltpu.SemaphoreType.DMA,
    ],
)
def _chase_sc(cell_hbm, cols_hbm, o_hbm, nbuf, mbuf, cbuf, obuf, sem):
    @pl.when(lax.axis_index("c") == 0)
    def _():
        pltpu.async_copy(cell_hbm, cbuf, sem).wait()
        c = cbuf[0]
        base = pl.multiple_of((c >> 7) << 7, 128)
        pltpu.async_copy(
            cols_hbm.at[pl.ds(0, 4), pl.ds(base, 128)], nbuf, sem).wait()
        lane = c & 127
        anchor = nbuf[2, lane]
        obuf[0] = nbuf[0, lane]
        obuf[8] = anchor
        abase = pl.multiple_of((anchor >> 7) << 7, 128)
        pltpu.async_copy(
            cols_hbm.at[pl.ds(0, 4), pl.ds(abase, 128)], mbuf, sem).wait()
        obuf[16] = mbuf[1, anchor & 127]
        pltpu.async_copy(obuf, o_hbm, sem).wait()


@functools.partial(
    pl.kernel,
    out_type=jax.ShapeDtypeStruct((2, D), jnp.float32),
    mesh=_vector_mesh,
    compiler_params=_cp,
    scratch_types=[
        pltpu.VMEM((24,), jnp.int32),
        pltpu.VMEM((1, D), jnp.float32),
        pltpu.VMEM((1, D), jnp.float32),
        pltpu.VMEM((D,), jnp.float32),
        pltpu.VMEM((1, D), jnp.float32),
        pltpu.SemaphoreType.DMA,
        pltpu.SemaphoreType.DMA,
    ],
)
def _interp_sc(x1_hbm, idx_hbm, X_hbm, o_hbm, iv, xa, xc, x1v, outv, s0, s1):
    core = lax.axis_index("c")
    sub = lax.axis_index("s")

    @pl.when((core == 0) & (sub == 0))
    def _():
        cp_x1 = pltpu.async_copy(x1_hbm, x1v, s0)
        pltpu.sync_copy(idx_hbm, iv)
        cp_xa = pltpu.async_copy(X_hbm.at[iv.at[pl.ds(0, 1)]], xa, s1)
        cp_x1.wait()

        @pl.loop(0, D, step=L)
        def _(i):
            outv[0, pl.ds(i, L)] = ALPHA * x1v[pl.ds(i, L)]

        cp_xa.wait()

        @pl.loop(0, D, step=L)
        def _(i):
            sl = pl.ds(i, L)
            outv[0, sl] = outv[0, sl] + BETA * xa[0, sl]

        pltpu.sync_copy(outv, o_hbm.at[pl.ds(0, 1)])

    @pl.when((core == 1) & (sub == 0))
    def _():
        pltpu.sync_copy(idx_hbm, iv)
        cp_xb = pltpu.async_copy(X_hbm.at[iv.at[pl.ds(8, 1)]], xa, s0)
        cp_xc = pltpu.async_copy(X_hbm.at[iv.at[pl.ds(16, 1)]], xc, s1)
        cp_xb.wait()

        @pl.loop(0, D, step=L)
        def _(i):
            outv[0, pl.ds(i, L)] = ALPHA * xa[0, pl.ds(i, L)]

        cp_xc.wait()

        @pl.loop(0, D, step=L)
        def _(i):
            sl = pl.ds(i, L)
            outv[0, sl] = outv[0, sl] + BETA * xc[0, sl]

        pltpu.sync_copy(outv, o_hbm.at[pl.ds(1, 1)])


def kernel(x1, x2, cell_ids, X, nns_idx, mnn_idx):
    if _COND:
        cell = cell_ids.astype(jnp.int32).reshape(1)
        cols = jnp.stack([nns_idx[:, _R_KN], nns_idx[:, _R_KNI],
                          mnn_idx[:, _R_KA], nns_idx[:, _R_KN]])
        idx = _chase_sc(cell, cols)
        return _interp_sc(x1, idx, X)
    else:
        return jnp.stack([x1, x2])


# speedup vs baseline: 2.0504x; 1.0080x over previous
import dataclasses
import functools

import jax
import jax.numpy as jnp
from jax import lax
from jax.experimental import pallas as pl
from jax.experimental.pallas import tpu as pltpu
from jax.experimental.pallas import tpu_sc as plsc

N, D, K, A = 20000, 2048, 16, 8
ALPHA = 0.9
BETA = 1.0 - ALPHA
APPLY_PROB = 0.9
NSIZE = 1
L = 16

_COND = True
_R_KN = 13
_R_KA = 1
_R_KNI = 6

_scalar_mesh = plsc.ScalarSubcoreMesh(axis_name="c", num_cores=2)
_vector_mesh = plsc.VectorSubcoreMesh(core_axis_name="c", subcore_axis_name="s")

_cp = pltpu.CompilerParams()
if "needs_layout_passes" in pltpu.CompilerParams.__dataclass_fields__:
    _cp = dataclasses.replace(_cp, needs_layout_passes=False)


@functools.partial(
    pl.kernel,
    out_type=jax.ShapeDtypeStruct((24,), jnp.int32),
    mesh=_scalar_mesh,
    scratch_types=[
        pltpu.SMEM((4, 128), jnp.int32),
        pltpu.SMEM((4, 128), jnp.int32),
        pltpu.SMEM((1,), jnp.int32),
        pltpu.SMEM((24,), jnp.int32),
        pltpu.SemaphoreType.DMA,
    ],
)
def _chase_sc(cell_hbm, cols_hbm, o_hbm, nbuf, mbuf, cbuf, obuf, sem):
    @pl.when(lax.axis_index("c") == 0)
    def _():
        pltpu.async_copy(cell_hbm, cbuf, sem).wait()
        c = cbuf[0]
        base = pl.multiple_of((c >> 7) << 7, 128)
        pltpu.async_copy(
            cols_hbm.at[pl.ds(0, 4), pl.ds(base, 128)], nbuf, sem).wait()
        lane = c & 127
        anchor = nbuf[2, lane]
        obuf[0] = nbuf[0, lane]
        obuf[8] = anchor
        abase = pl.multiple_of((anchor >> 7) << 7, 128)
        pltpu.async_copy(
            cols_hbm.at[pl.ds(0, 4), pl.ds(abase, 128)], mbuf, sem).wait()
        obuf[16] = mbuf[1, anchor & 127]
        pltpu.async_copy(obuf, o_hbm, sem).wait()


@functools.partial(
    pl.kernel,
    out_type=jax.ShapeDtypeStruct((2, D), jnp.float32),
    mesh=_vector_mesh,
    compiler_params=_cp,
    scratch_types=[
        pltpu.VMEM((24,), jnp.int32),
        pltpu.VMEM((1, D), jnp.float32),
        pltpu.VMEM((1, D), jnp.float32),
        pltpu.VMEM((D,), jnp.float32),
        pltpu.VMEM((1, D), jnp.float32),
        pltpu.SemaphoreType.DMA,
        pltpu.SemaphoreType.DMA,
    ],
)
def _interp_sc(x1_hbm, idx_hbm, X_hbm, o_hbm, iv, xa, xc, x1v, outv, s0, s1):
    core = lax.axis_index("c")
    sub = lax.axis_index("s")

    @pl.when((core == 0) & (sub == 0))
    def _():
        cp_x1 = pltpu.async_copy(x1_hbm, x1v, s0)
        pltpu.sync_copy(idx_hbm, iv)
        cp_xa = pltpu.async_copy(X_hbm.at[iv.at[pl.ds(0, 1)]], xa, s1)
        cp_x1.wait()
        cp_xa.wait()

        @pl.loop(0, D, step=L)
        def _(i):
            sl = pl.ds(i, L)
            outv[0, sl] = ALPHA * x1v[sl] + BETA * xa[0, sl]

        pltpu.sync_copy(outv, o_hbm.at[pl.ds(0, 1)])

    @pl.when((core == 1) & (sub == 0))
    def _():
        pltpu.sync_copy(idx_hbm, iv)
        cp_xb = pltpu.async_copy(X_hbm.at[iv.at[pl.ds(8, 1)]], xa, s0)
        cp_xc = pltpu.async_copy(X_hbm.at[iv.at[pl.ds(16, 1)]], xc, s1)
        cp_xb.wait()
        cp_xc.wait()

        @pl.loop(0, D, step=L)
        def _(i):
            sl = pl.ds(i, L)
            outv[0, sl] = ALPHA * xa[0, sl] + BETA * xc[0, sl]

        pltpu.sync_copy(outv, o_hbm.at[pl.ds(1, 1)])


def kernel(x1, x2, cell_ids, X, nns_idx, mnn_idx):
    if _COND:
        cell = cell_ids.astype(jnp.int32).reshape(1)
        cols = jnp.stack([nns_idx[:, _R_KN], nns_idx[:, _R_KNI],
                          mnn_idx[:, _R_KA], nns_idx[:, _R_KN]])
        idx = _chase_sc(cell, cols)
        return _interp_sc(x1, idx, X)
    else:
        return jnp.stack([x1, x2])


# speedup vs baseline: 2.2478x; 1.0963x over previous
import dataclasses
import functools

import jax
import jax.numpy as jnp
from jax import lax
from jax.experimental import pallas as pl
from jax.experimental.pallas import tpu as pltpu
from jax.experimental.pallas import tpu_sc as plsc

N, D, K, A = 20000, 2048, 16, 8
ALPHA = 0.9
BETA = 1.0 - ALPHA
APPLY_PROB = 0.9
NSIZE = 1
L = 16
W = 128
NB = (N + W - 1) // W
NPAD = NB * W - N

_COND = True
_R_KN = 13
_R_KA = 1
_R_KNI = 6

_vector_mesh = plsc.VectorSubcoreMesh(core_axis_name="c", subcore_axis_name="s")

_cp = pltpu.CompilerParams()
if "needs_layout_passes" in pltpu.CompilerParams.__dataclass_fields__:
    _cp = dataclasses.replace(_cp, needs_layout_passes=False)


@functools.partial(
    pl.kernel,
    out_type=jax.ShapeDtypeStruct((2, D), jnp.float32),
    mesh=_vector_mesh,
    compiler_params=_cp,
    scratch_types=[
        pltpu.VMEM((1, W), jnp.int32),
        pltpu.VMEM((1, 3 * W), jnp.int32),
        pltpu.VMEM((1, 3 * W), jnp.int32),
        pltpu.VMEM((L,), jnp.int32),
        pltpu.VMEM((L,), jnp.int32),
        pltpu.VMEM((1, D), jnp.float32),
        pltpu.VMEM((1, D), jnp.float32),
        pltpu.VMEM((D,), jnp.float32),
        pltpu.VMEM((1, D), jnp.float32),
        pltpu.SemaphoreType.DMA,
        pltpu.SemaphoreType.DMA,
        pltpu.SemaphoreType.DMA,
    ],
)
def _augment_sc(x1_hbm, cell_hbm, cols_hbm, X_hbm, o_hbm,
                cellv, crow, crow2, ib0, ib1, xa, xc, x1v, outv,
                s0, s1, s2):
    core = lax.axis_index("c")
    sub = lax.axis_index("s")
    zeros = jnp.zeros((L,), jnp.int32)

    @pl.when((core == 0) & (sub == 0))
    def _():
        cp_x1 = pltpu.async_copy(x1_hbm, x1v, s2)
        pltpu.sync_copy(cell_hbm, cellv)
        c = cellv[0, pl.ds(0, L)]
        ib0[...] = c >> 7
        pltpu.async_copy(cols_hbm.at[ib0.at[pl.ds(0, 1)]], crow, s0).wait()
        ib1[...] = plsc.load_gather(crow, [zeros, c & 127])
        cp_xa = pltpu.async_copy(X_hbm.at[ib1.at[pl.ds(0, 1)]], xa, s1)
        cp_x1.wait()
        cp_xa.wait()

        @pl.loop(0, D, step=L)
        def _(i):
            sl = pl.ds(i, L)
            outv[0, sl] = ALPHA * x1v[sl] + BETA * xa[0, sl]

        pltpu.sync_copy(outv, o_hbm.at[pl.ds(0, 1)])

    @pl.when((core == 1) & (sub == 0))
    def _():
        pltpu.sync_copy(cell_hbm, cellv)
        c = cellv[0, pl.ds(0, L)]
        ib0[...] = c >> 7
        pltpu.async_copy(cols_hbm.at[ib0.at[pl.ds(0, 1)]], crow, s0).wait()
        anchor = plsc.load_gather(crow, [zeros, (c & 127) + 2 * W])
        ib0[...] = anchor
        cp_xb = pltpu.async_copy(X_hbm.at[ib0.at[pl.ds(0, 1)]], xa, s1)
        ib1[...] = anchor >> 7
        cp_t2 = pltpu.async_copy(cols_hbm.at[ib1.at[pl.ds(0, 1)]], crow2, s0)
        cp_t2.wait()
        ib1[...] = plsc.load_gather(crow2, [zeros, (anchor & 127) + W])
        cp_xc = pltpu.async_copy(X_hbm.at[ib1.at[pl.ds(0, 1)]], xc, s2)
        cp_xb.wait()
        cp_xc.wait()

        @pl.loop(0, D, step=L)
        def _(i):
            sl = pl.ds(i, L)
            outv[0, sl] = ALPHA * xa[0, sl] + BETA * xc[0, sl]

        pltpu.sync_copy(outv, o_hbm.at[pl.ds(1, 1)])


def kernel(x1, x2, cell_ids, X, nns_idx, mnn_idx):
    if _COND:
        c = cell_ids.astype(jnp.int32)
        cell2d = jnp.full((1, W), c, jnp.int32)
        cols = jnp.stack([nns_idx[:, _R_KN], nns_idx[:, _R_KNI],
                          mnn_idx[:, _R_KA]])
        colsW = (jnp.pad(cols, ((0, 0), (0, NPAD)))
                 .reshape(3, NB, W).transpose(1, 0, 2).reshape(NB, 3 * W))
        return _augment_sc(x1, cell2d, colsW, X)
    else:
        return jnp.stack([x1, x2])


# speedup vs baseline: 2.3862x; 1.0616x over previous
import dataclasses
import functools

import jax
import jax.numpy as jnp
from jax import lax
from jax.experimental import pallas as pl
from jax.experimental.pallas import tpu as pltpu
from jax.experimental.pallas import tpu_sc as plsc

N, D, K, A = 20000, 2048, 16, 8
ALPHA = 0.9
BETA = 1.0 - ALPHA
APPLY_PROB = 0.9
NSIZE = 1
L = 16

_COND = True
_R_KN = 13
_R_KA = 1
_R_KNI = 6

_vector_mesh = plsc.VectorSubcoreMesh(core_axis_name="c", subcore_axis_name="s")

_cp = pltpu.CompilerParams()
if "needs_layout_passes" in pltpu.CompilerParams.__dataclass_fields__:
    _cp = dataclasses.replace(_cp, needs_layout_passes=False)


@functools.partial(
    pl.kernel,
    out_type=jax.ShapeDtypeStruct((2, D), jnp.float32),
    mesh=_vector_mesh,
    compiler_params=_cp,
    scratch_types=[
        pltpu.VMEM((1,), jnp.int32),
        pltpu.VMEM((L,), jnp.int32),
        pltpu.VMEM((L,), jnp.int32),
        pltpu.VMEM((1, D), jnp.float32),
        pltpu.VMEM((1, D), jnp.float32),
        pltpu.VMEM((D,), jnp.float32),
        pltpu.VMEM((1, D), jnp.float32),
        pltpu.SemaphoreType.DMA,
        pltpu.SemaphoreType.DMA,
        pltpu.SemaphoreType.DMA,
    ],
)
def _augment_sc(x1_hbm, cell_hbm, coln_hbm, coln2_hbm, colm_hbm, X_hbm, o_hbm,
                cellv, ib0, ib1, xa, xc, x1v, outv, s0, s1, s2):
    core = lax.axis_index("c")
    sub = lax.axis_index("s")
    zeros = jnp.zeros((L,), jnp.int32)

    @pl.when((core == 0) & (sub == 0))
    def _():
        cp_x1 = pltpu.async_copy(x1_hbm, x1v, s2)
        pltpu.sync_copy(cell_hbm, cellv)
        c = plsc.load_gather(cellv, [zeros])
        pltpu.async_copy(coln_hbm.at[c], ib0, s0).wait()
        cp_xa = pltpu.async_copy(X_hbm.at[ib0.at[pl.ds(0, 1)]], xa, s1)
        cp_x1.wait()
        cp_xa.wait()

        @pl.loop(0, D, step=L)
        def _(i):
            sl = pl.ds(i, L)
            outv[0, sl] = ALPHA * x1v[sl] + BETA * xa[0, sl]

        pltpu.sync_copy(outv, o_hbm.at[pl.ds(0, 1)])

    @pl.when((core == 1) & (sub == 0))
    def _():
        pltpu.sync_copy(cell_hbm, cellv)
        c = plsc.load_gather(cellv, [zeros])
        pltpu.async_copy(colm_hbm.at[c], ib0, s0).wait()
        cp_xb = pltpu.async_copy(X_hbm.at[ib0.at[pl.ds(0, 1)]], xa, s1)
        anchor = ib0[pl.ds(0, L)]
        pltpu.async_copy(coln2_hbm.at[anchor], ib1, s0).wait()
        cp_xc = pltpu.async_copy(X_hbm.at[ib1.at[pl.ds(0, 1)]], xc, s2)
        cp_xb.wait()
        cp_xc.wait()

        @pl.loop(0, D, step=L)
        def _(i):
            sl = pl.ds(i, L)
            outv[0, sl] = ALPHA * xa[0, sl] + BETA * xc[0, sl]

        pltpu.sync_copy(outv, o_hbm.at[pl.ds(1, 1)])


def kernel(x1, x2, cell_ids, X, nns_idx, mnn_idx):
    if _COND:
        cell = cell_ids.astype(jnp.int32).reshape(1)
        return _augment_sc(x1, cell, nns_idx[:, _R_KN], nns_idx[:, _R_KNI],
                           mnn_idx[:, _R_KA], X)
    else:
        return jnp.stack([x1, x2])
